# Initial kernel scaffold; baseline (speedup 1.0000x reference)
import jax
import jax.numpy as jnp
from jax.experimental import pallas as pl


def kernel(inputs, table):
    raise NotImplementedError("write your pallas kernel here")


# speedup vs baseline: 5.5499x; 5.5499x over previous
import jax
import jax.numpy as jnp
from jax.experimental import pallas as pl

_UNITS = 768
_SCALE = _UNITS ** 0.5
_ROWS = 512


def _bcast_kernel(table_ref, out_ref):
    scaled = table_ref[...] * _SCALE
    out_ref[...] = jnp.broadcast_to(scaled[None, :, :], out_ref.shape)


def kernel(inputs, table):
    n, t = inputs.shape
    units = table.shape[1]
    grid = (t // _ROWS,)
    out = pl.pallas_call(
        _bcast_kernel,
        grid=grid,
        in_specs=[pl.BlockSpec((_ROWS, units), lambda i: (i, 0))],
        out_specs=pl.BlockSpec((n, _ROWS, units), lambda i: (0, i, 0)),
        out_shape=jax.ShapeDtypeStruct((n, t, units), table.dtype),
    )(table)
    return out
